# baseline (speedup 1.0000x reference)
import functools

import jax
import jax.numpy as jnp
from jax import lax
from jax.experimental import pallas as pl
from jax.experimental.pallas import tpu as pltpu
from jax.experimental.pallas import tpu_sc as plsc

NUM_ENT = 1000000
NUM_REL = 1000
D = 64
B = 16384
NC = 2
NS = 16
L = 16
NW = NC * NS
BPW = B // NW
CHUNK = 128
NCHUNK = BPW // CHUNK
GROUPS = CHUNK // L


def _tec_body(heads_hbm, rels_hbm, tails_hbm, ent_hbm, rel_hbm, norm_hbm,
              out_hbm,
              idxh_v, idxt_v, idxr_v, h_v, t_v, r_v, n_v, scores_v, sem):
    wid = lax.axis_index("s") * NC + lax.axis_index("c")
    base = wid * BPW

    def do_chunk(c, _):
        off = base + c * CHUNK
        pltpu.sync_copy(heads_hbm.at[pl.ds(off, CHUNK)], idxh_v)
        pltpu.sync_copy(tails_hbm.at[pl.ds(off, CHUNK)], idxt_v)
        pltpu.sync_copy(rels_hbm.at[pl.ds(off, CHUNK)], idxr_v)
        cp_h = pltpu.async_copy(ent_hbm.at[idxh_v], h_v, sem)
        cp_t = pltpu.async_copy(ent_hbm.at[idxt_v], t_v, sem)
        cp_r = pltpu.async_copy(rel_hbm.at[idxr_v], r_v, sem)
        cp_n = pltpu.async_copy(norm_hbm.at[idxr_v], n_v, sem)
        cp_h.wait()
        cp_t.wait()
        cp_r.wait()
        cp_n.wait()

        def do_group(g, _):
            vec = jnp.zeros((L,), jnp.float32)
            lane = lax.broadcasted_iota(jnp.int32, (L,), 0)
            for j in range(L):
                row = g * L + j
                h = [h_v[row, pl.ds(d * L, L)] for d in range(D // L)]
                t = [t_v[row, pl.ds(d * L, L)] for d in range(D // L)]
                r = [r_v[row, pl.ds(d * L, L)] for d in range(D // L)]
                n = [n_v[row, pl.ds(d * L, L)] for d in range(D // L)]
                dh = jnp.sum((h[0] * n[0] + h[1] * n[1])
                             + (h[2] * n[2] + h[3] * n[3]))
                dt = jnp.sum((t[0] * n[0] + t[1] * n[1])
                             + (t[2] * n[2] + t[3] * n[3]))
                parts = [jnp.abs((h[d] - t[d]) + r[d] + (dt - dh) * n[d])
                         for d in range(D // L)]
                s = jnp.sum((parts[0] + parts[1]) + (parts[2] + parts[3]))
                vec = jnp.where(lane == j, s, vec)
            scores_v[pl.ds(c * CHUNK + g * L, L)] = vec
            return 0

        lax.fori_loop(0, GROUPS, do_group, 0)
        return 0

    lax.fori_loop(0, NCHUNK, do_chunk, 0)
    pltpu.sync_copy(scores_v, out_hbm.at[pl.ds(base, BPW)])


@jax.jit
def kernel(heads, rels, tails, ent_embs, rel_embs, norm_vector):
    mesh = plsc.VectorSubcoreMesh(core_axis_name="c", subcore_axis_name="s",
                                  num_cores=NC, num_subcores=NS)
    run = pl.kernel(
        _tec_body,
        out_type=jax.ShapeDtypeStruct((B,), jnp.float32),
        mesh=mesh,
        compiler_params=pltpu.CompilerParams(needs_layout_passes=False,
                                             use_tc_tiling_on_sc=False),
        scratch_types=[
            pltpu.VMEM((CHUNK,), jnp.int32),
            pltpu.VMEM((CHUNK,), jnp.int32),
            pltpu.VMEM((CHUNK,), jnp.int32),
            pltpu.VMEM((CHUNK, D), jnp.float32),
            pltpu.VMEM((CHUNK, D), jnp.float32),
            pltpu.VMEM((CHUNK, D), jnp.float32),
            pltpu.VMEM((CHUNK, D), jnp.float32),
            pltpu.VMEM((BPW,), jnp.float32),
            pltpu.SemaphoreType.DMA,
        ],
    )
    return run(heads, rels, tails, ent_embs, rel_embs, norm_vector)
